# Initial kernel scaffold; baseline (speedup 1.0000x reference)
import jax
import jax.numpy as jnp
from jax.experimental import pallas as pl


def kernel(x, edge_index, W1, b1, W2, b2):
    raise NotImplementedError("write your pallas kernel here")


# speedup vs baseline: 22.7430x; 22.7430x over previous
import functools

import jax
import jax.numpy as jnp
from jax import lax
from jax.experimental import pallas as pl
from jax.experimental.pallas import tpu as pltpu
from jax.experimental.pallas import tpu_sc as plsc

N = 10000
D = 128
H = 64
O = 2
E = 320000

NC = 2
NS = 16
NW = NC * NS
WIN = 128
NWIN = 80
EPAD = NW * NWIN * WIN
NPAD = 10240
RPT = NPAD // NS

_mesh = plsc.VectorSubcoreMesh(core_axis_name="c", subcore_axis_name="s")


def _deg_body(dst_hbm, ones_hbm, zeros_hbm, out_hbm, dst_v, ones_v, z_v, deg_sh):
    c = lax.axis_index("c")
    s = lax.axis_index("s")
    wid = s * NC + c
    pltpu.sync_copy(zeros_hbm, z_v)
    pltpu.sync_copy(z_v, deg_sh.at[pl.ds(s * RPT, RPT)])
    pltpu.sync_copy(ones_hbm, ones_v)
    pltpu.sync_copy(dst_hbm.at[wid], dst_v)
    plsc.subcore_barrier()

    def body(j, carry):
        pltpu.sync_copy(ones_v, deg_sh.at[dst_v.at[j]], add=True)
        return carry

    lax.fori_loop(0, NWIN, body, 0, unroll=4)
    plsc.subcore_barrier()
    pltpu.sync_copy(deg_sh.at[pl.ds(s * RPT, RPT)],
                    out_hbm.at[c, pl.ds(s * RPT, RPT)])


def _make_agg(width):
    def body(table_hbm, src_hbm, dst_hbm, zeros_hbm, out_hbm,
             src_v, dst_v, buf0, buf1, z_v, acc_sh, sem0, sem1):
        c = lax.axis_index("c")
        s = lax.axis_index("s")
        wid = s * NC + c
        pltpu.sync_copy(zeros_hbm, z_v)
        for t in range(RPT // WIN):
            pltpu.sync_copy(z_v, acc_sh.at[pl.ds(s * RPT + t * WIN, WIN)])
        pltpu.sync_copy(src_hbm.at[wid], src_v)
        pltpu.sync_copy(dst_hbm.at[wid], dst_v)
        plsc.subcore_barrier()

        pltpu.async_copy(table_hbm.at[src_v.at[0]], buf0, sem0)

        def body(i, carry):
            w = 2 * i
            pltpu.make_async_copy(table_hbm.at[src_v.at[w]], buf0, sem0).wait()
            pltpu.async_copy(table_hbm.at[src_v.at[w + 1]], buf1, sem1)
            pltpu.sync_copy(buf0, acc_sh.at[dst_v.at[w]], add=True)
            pltpu.make_async_copy(table_hbm.at[src_v.at[w + 1]], buf1, sem1).wait()

            @pl.when(i < NWIN // 2 - 1)
            def _():
                pltpu.async_copy(table_hbm.at[src_v.at[w + 2]], buf0, sem0)

            pltpu.sync_copy(buf1, acc_sh.at[dst_v.at[w + 1]], add=True)
            return carry

        lax.fori_loop(0, NWIN // 2, body, 0)
        plsc.subcore_barrier()
        for t in range(RPT // WIN):
            r = s * RPT + t * WIN
            pltpu.sync_copy(acc_sh.at[pl.ds(r, WIN)],
                            out_hbm.at[c, pl.ds(r, WIN)])

    return pl.kernel(
        body,
        out_type=jax.ShapeDtypeStruct((NC, NPAD, width), jnp.float32),
        mesh=_mesh,
        compiler_params=pltpu.CompilerParams(use_tc_tiling_on_sc=False),
        scratch_types=[
            pltpu.VMEM((NWIN, WIN), jnp.int32),
            pltpu.VMEM((NWIN, WIN), jnp.int32),
            pltpu.VMEM((WIN, width), jnp.float32),
            pltpu.VMEM((WIN, width), jnp.float32),
            pltpu.VMEM((WIN, width), jnp.float32),
            pltpu.VMEM_SHARED((NPAD, width), jnp.float32),
            pltpu.SemaphoreType.DMA,
            pltpu.SemaphoreType.DMA,
        ],
    )


_deg_kernel = pl.kernel(
    _deg_body,
    out_type=jax.ShapeDtypeStruct((NC, NPAD), jnp.float32),
    mesh=_mesh,
    scratch_types=[
        pltpu.VMEM((NWIN, WIN), jnp.int32),
        pltpu.VMEM((WIN,), jnp.float32),
        pltpu.VMEM((RPT,), jnp.float32),
        pltpu.VMEM_SHARED((NPAD,), jnp.float32),
    ],
)

W2P = 16

_agg64 = _make_agg(H)
_agg2 = _make_agg(W2P)


def _mm1_body(degp_ref, x_ref, w1_ref, xws_ref, dis_ref):
    deg = degp_ref[0, :] + degp_ref[1, :] + 1.0
    dis = lax.rsqrt(deg)
    dis_ref[...] = dis
    xw = jnp.dot(x_ref[...], w1_ref[...], preferred_element_type=jnp.float32)
    xws_ref[...] = xw * dis[:, None]


def _mm2_body(aggp_ref, xws_ref, dis_ref, b1_ref, w2_ref, ps_ref):
    agg = aggp_ref[0] + aggp_ref[1]
    dis = dis_ref[...]
    h = jnp.maximum(dis[:, None] * (agg + xws_ref[...]) + b1_ref[...], 0.0)
    ps = dis[:, None] * jnp.dot(h, w2_ref[...], preferred_element_type=jnp.float32)
    rows = lax.broadcasted_iota(jnp.int32, ps.shape, 0)
    ps_ref[...] = jnp.where(rows < N, ps, 0.0)


def _fin_body(aggp_ref, ps_ref, dis_ref, b2_ref, out_ref):
    agg = aggp_ref[0] + aggp_ref[1] + ps_ref[...]
    out_ref[...] = dis_ref[...][:, None] * agg[:, :O] + b2_ref[...]


def kernel(x, edge_index, W1, b1, W2, b2):
    f32 = jnp.float32
    pad = jnp.full((EPAD - E,), N, dtype=jnp.int32)
    srcp = jnp.concatenate([edge_index[0], pad]).reshape(NW, NWIN, WIN)
    dstp = jnp.concatenate([edge_index[1], pad]).reshape(NW, NWIN, WIN)
    x_pad = jnp.pad(x, ((0, NPAD - N), (0, 0)))
    ones_w = jnp.ones((WIN,), f32)
    zeros_r = jnp.zeros((RPT,), f32)
    zeros_h = jnp.zeros((WIN, H), f32)
    zeros_o = jnp.zeros((WIN, W2P), f32)
    W2p = jnp.pad(W2, ((0, 0), (0, W2P - O)))

    degp = _deg_kernel(dstp, ones_w, zeros_r)

    xws, dis = pl.pallas_call(
        _mm1_body,
        out_shape=[jax.ShapeDtypeStruct((NPAD, H), f32),
                   jax.ShapeDtypeStruct((NPAD,), f32)],
    )(degp, x_pad, W1)

    agg1p = _agg64(xws, srcp, dstp, zeros_h)

    ps = pl.pallas_call(
        _mm2_body,
        out_shape=jax.ShapeDtypeStruct((NPAD, W2P), f32),
    )(agg1p, xws, dis, b1, W2p)

    agg2p = _agg2(ps, srcp, dstp, zeros_o)

    out = pl.pallas_call(
        _fin_body,
        out_shape=jax.ShapeDtypeStruct((NPAD, O), f32),
    )(agg2p, ps, dis, b2)
    return out[:N]


# speedup vs baseline: 26.1665x; 1.1505x over previous
import functools

import jax
import jax.numpy as jnp
from jax import lax
from jax.experimental import pallas as pl
from jax.experimental.pallas import tpu as pltpu
from jax.experimental.pallas import tpu_sc as plsc

N = 10000
D = 128
H = 64
O = 2
E = 320000

NC = 2
NS = 16
NW = NC * NS
WIN = 128
NWIN = 80
EPAD = NW * NWIN * WIN
NPAD = 10240
RPT = NPAD // NS

_mesh = plsc.VectorSubcoreMesh(core_axis_name="c", subcore_axis_name="s")


def _deg_body(dst_hbm, ones_hbm, zeros_hbm, out_hbm, dst_v, ones_v, z_v, deg_sh):
    c = lax.axis_index("c")
    s = lax.axis_index("s")
    wid = s * NC + c
    pltpu.sync_copy(zeros_hbm, z_v)
    pltpu.sync_copy(z_v, deg_sh.at[pl.ds(s * RPT, RPT)])
    pltpu.sync_copy(ones_hbm, ones_v)
    pltpu.sync_copy(dst_hbm.at[wid], dst_v)
    plsc.subcore_barrier()

    def body(j, carry):
        pltpu.sync_copy(ones_v, deg_sh.at[dst_v.at[j]], add=True)
        return carry

    lax.fori_loop(0, NWIN, body, 0, unroll=4)
    plsc.subcore_barrier()
    pltpu.sync_copy(deg_sh.at[pl.ds(s * RPT, RPT)],
                    out_hbm.at[c, pl.ds(s * RPT, RPT)])


def _make_agg(width):
    NBUF = 4

    def body(table_hbm, src_hbm, dst_hbm, zeros_hbm, out_hbm,
             src_v, dst_v, bufs, z_v, acc_sh, gsems, ssems):
        c = lax.axis_index("c")
        s = lax.axis_index("s")
        wid = c * NS + s
        pltpu.sync_copy(zeros_hbm, z_v)
        for t in range(RPT // WIN):
            pltpu.sync_copy(z_v, acc_sh.at[pl.ds(s * RPT + t * WIN, WIN)])
        pltpu.sync_copy(src_hbm.at[wid], src_v)
        pltpu.sync_copy(dst_hbm.at[wid], dst_v)
        plsc.subcore_barrier()

        def gather(w, k):
            pltpu.async_copy(table_hbm.at[src_v.at[w]], bufs.at[k], gsems.at[k])

        def gwait(w, k):
            pltpu.make_async_copy(table_hbm.at[src_v.at[w]], bufs.at[k],
                                  gsems.at[k]).wait()

        def scat(w, k):
            pltpu.async_copy(bufs.at[k], acc_sh.at[dst_v.at[w]], ssems.at[k],
                             add=True)

        def swait(w, k):
            pltpu.make_async_copy(bufs.at[k], acc_sh.at[dst_v.at[w]],
                                  ssems.at[k]).wait()

        gather(0, 0)
        gather(1, 1)

        def loop_body(i, carry):
            for k in range(NBUF):
                w = NBUF * i + k
                gwait(w, k)
                scat(w, k)
                k2 = (k + 2) % NBUF
                w_prev = w - 2

                @pl.when(w_prev >= 0)
                def _():
                    swait(w_prev, k2)

                @pl.when(w + 2 < NWIN)
                def _():
                    gather(w + 2, k2)
            return carry

        lax.fori_loop(0, NWIN // NBUF, loop_body, 0)
        swait(NWIN - 2, (NWIN - 2) % NBUF)
        swait(NWIN - 1, (NWIN - 1) % NBUF)
        plsc.subcore_barrier()
        for t in range(RPT // WIN):
            r = s * RPT + t * WIN
            pltpu.sync_copy(acc_sh.at[pl.ds(r, WIN)],
                            out_hbm.at[c, pl.ds(r, WIN)])

    return pl.kernel(
        body,
        out_type=jax.ShapeDtypeStruct((NC, NPAD, width), jnp.float32),
        mesh=_mesh,
        compiler_params=pltpu.CompilerParams(use_tc_tiling_on_sc=False),
        scratch_types=[
            pltpu.VMEM((NWIN, WIN), jnp.int32),
            pltpu.VMEM((NWIN, WIN), jnp.int32),
            pltpu.VMEM((NBUF, WIN, width), jnp.float32),
            pltpu.VMEM((WIN, width), jnp.float32),
            pltpu.VMEM_SHARED((NPAD, width), jnp.float32),
            pltpu.SemaphoreType.DMA((NBUF,)),
            pltpu.SemaphoreType.DMA((NBUF,)),
        ],
    )


_deg_kernel = pl.kernel(
    _deg_body,
    out_type=jax.ShapeDtypeStruct((NC, NPAD), jnp.float32),
    mesh=_mesh,
    scratch_types=[
        pltpu.VMEM((NWIN, WIN), jnp.int32),
        pltpu.VMEM((WIN,), jnp.float32),
        pltpu.VMEM((RPT,), jnp.float32),
        pltpu.VMEM_SHARED((NPAD,), jnp.float32),
    ],
)

W2P = 16

_agg64 = _make_agg(H)
_agg2 = _make_agg(W2P)


def _mm1_body(degp_ref, x_ref, w1_ref, xws_ref, dis_ref):
    deg = degp_ref[0, :] + degp_ref[1, :] + 1.0
    dis = lax.rsqrt(deg)
    dis_ref[...] = dis
    xw = jnp.dot(x_ref[...], w1_ref[...], preferred_element_type=jnp.float32)
    xws_ref[...] = xw * dis[:, None]


def _mm2_body(aggp_ref, xws_ref, dis_ref, b1_ref, w2_ref, ps_ref):
    agg = aggp_ref[0] + aggp_ref[1]
    dis = dis_ref[...]
    h = jnp.maximum(dis[:, None] * (agg + xws_ref[...]) + b1_ref[...], 0.0)
    ps = dis[:, None] * jnp.dot(h, w2_ref[...], preferred_element_type=jnp.float32)
    rows = lax.broadcasted_iota(jnp.int32, ps.shape, 0)
    ps_ref[...] = jnp.where(rows < N, ps, 0.0)


def _fin_body(aggp_ref, ps_ref, dis_ref, b2_ref, out_ref):
    agg = aggp_ref[0] + aggp_ref[1] + ps_ref[...]
    out_ref[...] = dis_ref[...][:, None] * agg[:, :O] + b2_ref[...]


def kernel(x, edge_index, W1, b1, W2, b2):
    f32 = jnp.float32
    pad = jnp.full((EPAD - E,), N, dtype=jnp.int32)
    srcp = jnp.concatenate([edge_index[0], pad]).reshape(NW, NWIN, WIN)
    dstp = jnp.concatenate([edge_index[1], pad]).reshape(NW, NWIN, WIN)
    x_pad = jnp.pad(x, ((0, NPAD - N), (0, 0)))
    ones_w = jnp.ones((WIN,), f32)
    zeros_r = jnp.zeros((RPT,), f32)
    zeros_h = jnp.zeros((WIN, H), f32)
    zeros_o = jnp.zeros((WIN, W2P), f32)
    W2p = jnp.pad(W2, ((0, 0), (0, W2P - O)))

    degp = _deg_kernel(dstp, ones_w, zeros_r)

    xws, dis = pl.pallas_call(
        _mm1_body,
        out_shape=[jax.ShapeDtypeStruct((NPAD, H), f32),
                   jax.ShapeDtypeStruct((NPAD,), f32)],
    )(degp, x_pad, W1)

    agg1p = _agg64(xws, srcp, dstp, zeros_h)

    ps = pl.pallas_call(
        _mm2_body,
        out_shape=jax.ShapeDtypeStruct((NPAD, W2P), f32),
    )(agg1p, xws, dis, b1, W2p)

    agg2p = _agg2(ps, srcp, dstp, zeros_o)

    out = pl.pallas_call(
        _fin_body,
        out_shape=jax.ShapeDtypeStruct((NPAD, O), f32),
    )(agg2p, ps, dis, b2)
    return out[:N]


# speedup vs baseline: 44.8201x; 1.7129x over previous
import functools

import jax
import jax.numpy as jnp
from jax import lax
from jax.experimental import pallas as pl
from jax.experimental.pallas import tpu as pltpu
from jax.experimental.pallas import tpu_sc as plsc

N = 10000
D = 128
H = 64
O = 2
E = 320000

NC = 2
NS = 16
NW = NC * NS
WIN = 128
NWIN = 80
EPAD = NW * NWIN * WIN
NPAD = 10240
RPT = NPAD // NS

_mesh = plsc.VectorSubcoreMesh(core_axis_name="c", subcore_axis_name="s")


def _unpack_edges(packed_v, src_v, dst_v, nrows):
    def row(j, carry):
        for l in range(WIN // 16):
            v = packed_v[j, pl.ds(16 * l, 16)]
            src_v[j, pl.ds(16 * l, 16)] = lax.bitwise_and(v, 0x3FFF)
            dst_v[j, pl.ds(16 * l, 16)] = lax.shift_right_logical(v, 14)
        return carry

    lax.fori_loop(0, nrows, row, 0)


def _deg_body(edg_hbm, ones_hbm, zeros_hbm, out_hbm,
              packed_v, dst_v, ones_v, z_v, deg_sh):
    c = lax.axis_index("c")
    s = lax.axis_index("s")
    wid = s * NC + c
    pltpu.sync_copy(zeros_hbm, z_v)
    pltpu.sync_copy(z_v, deg_sh.at[pl.ds(s * RPT, RPT)])
    pltpu.sync_copy(ones_hbm, ones_v)
    pltpu.sync_copy(edg_hbm.at[wid], packed_v)

    def row(j, carry):
        for l in range(WIN // 16):
            v = packed_v[j, pl.ds(16 * l, 16)]
            dst_v[j, pl.ds(16 * l, 16)] = lax.shift_right_logical(v, 14)
        return carry

    lax.fori_loop(0, NWIN, row, 0)
    plsc.subcore_barrier()

    def body(j, carry):
        pltpu.sync_copy(ones_v, deg_sh.at[dst_v.at[j]], add=True)
        return carry

    lax.fori_loop(0, NWIN, body, 0, unroll=4)
    plsc.subcore_barrier()
    pltpu.sync_copy(deg_sh.at[pl.ds(s * RPT, RPT)],
                    out_hbm.at[c, pl.ds(s * RPT, RPT)])


def _make_agg(sw, col_split):
    NBUF = 4
    CH = 2 if col_split else 1
    T = CH * NWIN

    def body(table_hbm, edg_hbm, zeros_hbm, out_hbm,
             packed_v, src_v, dst_v, bufs, z_v, table_sh, acc_sh, gsems, ssems):
        c = lax.axis_index("c")
        s = lax.axis_index("s")
        pltpu.sync_copy(zeros_hbm, z_v)
        for t in range(RPT // WIN):
            r = s * RPT + t * WIN
            pltpu.sync_copy(z_v, acc_sh.at[pl.ds(r, WIN)])
            if col_split:
                pltpu.sync_copy(table_hbm.at[c, pl.ds(r, WIN)], bufs.at[0])
            else:
                pltpu.sync_copy(table_hbm.at[pl.ds(r, WIN)], bufs.at[0])
            pltpu.sync_copy(bufs.at[0], table_sh.at[pl.ds(r, WIN)])
        for ch in range(CH):
            chunk = s * CH + ch if col_split else c * NS + s
            pltpu.sync_copy(edg_hbm.at[chunk],
                            packed_v.at[pl.ds(ch * NWIN, NWIN)])
        _unpack_edges(packed_v, src_v, dst_v, T)
        plsc.subcore_barrier()

        def gather(w, k):
            pltpu.async_copy(table_sh.at[src_v.at[w]], bufs.at[k], gsems.at[k])

        def gwait(w, k):
            pltpu.make_async_copy(table_sh.at[src_v.at[w]], bufs.at[k],
                                  gsems.at[k]).wait()

        def scat(w, k):
            pltpu.async_copy(bufs.at[k], acc_sh.at[dst_v.at[w]], ssems.at[k],
                             add=True)

        def swait(w, k):
            pltpu.make_async_copy(bufs.at[k], acc_sh.at[dst_v.at[w]],
                                  ssems.at[k]).wait()

        gather(0, 0)
        gather(1, 1)

        def loop_body(i, carry):
            for k in range(NBUF):
                w = NBUF * i + k
                gwait(w, k)
                scat(w, k)
                k2 = (k + 2) % NBUF
                w_prev = w - 2

                @pl.when(w_prev >= 0)
                def _():
                    swait(w_prev, k2)

                @pl.when(w + 2 < T)
                def _():
                    gather(w + 2, k2)
            return carry

        lax.fori_loop(0, T // NBUF, loop_body, 0)
        swait(T - 2, (T - 2) % NBUF)
        swait(T - 1, (T - 1) % NBUF)
        plsc.subcore_barrier()
        for t in range(RPT // WIN):
            r = s * RPT + t * WIN
            pltpu.sync_copy(acc_sh.at[pl.ds(r, WIN)],
                            out_hbm.at[c, pl.ds(r, WIN)])

    table_type = ((NC, NPAD, sw) if col_split else (NPAD, sw))
    return pl.kernel(
        body,
        out_type=jax.ShapeDtypeStruct((NC, NPAD, sw), jnp.float32),
        mesh=_mesh,
        compiler_params=pltpu.CompilerParams(use_tc_tiling_on_sc=False),
        scratch_types=[
            pltpu.VMEM((T, WIN), jnp.int32),
            pltpu.VMEM((T, WIN), jnp.int32),
            pltpu.VMEM((T, WIN), jnp.int32),
            pltpu.VMEM((NBUF, WIN, sw), jnp.float32),
            pltpu.VMEM((WIN, sw), jnp.float32),
            pltpu.VMEM_SHARED((NPAD, sw), jnp.float32),
            pltpu.VMEM_SHARED((NPAD, sw), jnp.float32),
            pltpu.SemaphoreType.DMA((NBUF,)),
            pltpu.SemaphoreType.DMA((NBUF,)),
        ],
    )


_deg_kernel = pl.kernel(
    _deg_body,
    out_type=jax.ShapeDtypeStruct((NC, NPAD), jnp.float32),
    mesh=_mesh,
    scratch_types=[
        pltpu.VMEM((NWIN, WIN), jnp.int32),
        pltpu.VMEM((NWIN, WIN), jnp.int32),
        pltpu.VMEM((WIN,), jnp.float32),
        pltpu.VMEM((RPT,), jnp.float32),
        pltpu.VMEM_SHARED((NPAD,), jnp.float32),
    ],
)

W2P = 16

_agg64 = _make_agg(H // NC, True)
_agg2 = _make_agg(W2P, False)


def _mm1_body(degp_ref, x_ref, w1_ref, xwsplit_ref, xws_ref, dis_ref):
    deg = degp_ref[0, :] + degp_ref[1, :] + 1.0
    dis = lax.rsqrt(deg)
    dis_ref[...] = dis
    xw = jnp.dot(x_ref[...], w1_ref[...], preferred_element_type=jnp.float32)
    xws = xw * dis[:, None]
    xws_ref[...] = xws
    xwsplit_ref[0] = xws[:, : H // NC]
    xwsplit_ref[1] = xws[:, H // NC :]


def _mm2_body(aggp_ref, xws_ref, dis_ref, b1_ref, w2_ref, ps_ref):
    agg = jnp.concatenate([aggp_ref[0], aggp_ref[1]], axis=1)
    dis = dis_ref[...]
    h = jnp.maximum(dis[:, None] * (agg + xws_ref[...]) + b1_ref[...], 0.0)
    ps = dis[:, None] * jnp.dot(h, w2_ref[...], preferred_element_type=jnp.float32)
    rows = lax.broadcasted_iota(jnp.int32, ps.shape, 0)
    ps_ref[...] = jnp.where(rows < N, ps, 0.0)


def _fin_body(aggp_ref, ps_ref, dis_ref, b2_ref, out_ref):
    agg = aggp_ref[0] + aggp_ref[1] + ps_ref[...]
    out_ref[...] = dis_ref[...][:, None] * agg[:, :O] + b2_ref[...]


def kernel(x, edge_index, W1, b1, W2, b2):
    f32 = jnp.float32
    pad = jnp.full((EPAD - E,), N + (N << 14), dtype=jnp.int32)
    packed = edge_index[0] + (edge_index[1] << 14)
    edg = jnp.concatenate([packed, pad]).reshape(NW, NWIN, WIN)
    x_pad = jnp.pad(x, ((0, NPAD - N), (0, 0)))
    ones_w = jnp.ones((WIN,), f32)
    zeros_r = jnp.zeros((RPT,), f32)
    zeros_hs = jnp.zeros((WIN, H // NC), f32)
    zeros_o = jnp.zeros((WIN, W2P), f32)
    W2p = jnp.pad(W2, ((0, 0), (0, W2P - O)))

    degp = _deg_kernel(edg, ones_w, zeros_r)

    xwsplit, xws, dis = pl.pallas_call(
        _mm1_body,
        out_shape=[jax.ShapeDtypeStruct((NC, NPAD, H // NC), f32),
                   jax.ShapeDtypeStruct((NPAD, H), f32),
                   jax.ShapeDtypeStruct((NPAD,), f32)],
    )(degp, x_pad, W1)

    agg1p = _agg64(xwsplit, edg, zeros_hs)

    ps = pl.pallas_call(
        _mm2_body,
        out_shape=jax.ShapeDtypeStruct((NPAD, W2P), f32),
    )(agg1p, xws, dis, b1, W2p)

    agg2p = _agg2(ps, edg, zeros_o)

    out = pl.pallas_call(
        _fin_body,
        out_shape=jax.ShapeDtypeStruct((NPAD, O), f32),
    )(agg2p, ps, dis, b2)
    return out[:N]
